# baseline (speedup 1.0000x reference)
import functools

import jax
import jax.numpy as jnp
from jax import lax
from jax.experimental import pallas as pl
from jax.experimental.pallas import tpu as pltpu
from jax.experimental.pallas import tpu_sc as plsc

_NUM_USERS = 2000
_NUM_ITEMS = 8000
_N = _NUM_USERS + _NUM_ITEMS
_EMB = 128
_NNZ = 320000

_NC = 2
_NS = 16
_NW = _NC * _NS
_K = 128
_BB = 16
_NB = 5
_CPW = _BB * _NB
_NNZ_PAD = _NW * _CPW * _K
_RPT = 624
_RTAIL = _N - _RPT * _NS
_ZROWS = _RPT // 3


def _splat(vv, e):
    idx = jnp.full((16, 1), e, jnp.int32)
    dn = lax.GatherDimensionNumbers(offset_dims=(), collapsed_slice_dims=(0,),
                                    start_index_map=(0,))
    return lax.gather(vv, idx, dn, (1,),
                      mode=lax.GatherScatterMode.PROMISE_IN_BOUNDS)


def _sc_spmv_body(ego_hbm, ecol_hbm, erow_hbm, eval_hbm, out_hbm,
                  cslab, rslab, vslab, bufs, acc_sh, gsems, esem):
    cid = lax.axis_index("c")
    sid = lax.axis_index("s")
    wid = cid * _NS + sid

    zero = jnp.zeros((16,), jnp.float32)

    def zrow(i, carry):
        for d in range(_EMB // 16):
            bufs[0, i, pl.ds(d * 16, 16)] = zero
        return carry

    lax.fori_loop(0, _K, zrow, 0)
    zsrc = bufs.at[0]
    zstart = pl.multiple_of(sid * _RPT, 8)
    for k in range(_RPT // _K):
        pltpu.sync_copy(zsrc, acc_sh.at[pl.ds(zstart + k * _K, _K)])
    pltpu.sync_copy(zsrc.at[pl.ds(0, _RPT % _K)],
                    acc_sh.at[pl.ds(zstart + _RPT - _RPT % _K, _RPT % _K)])

    @pl.when(sid == _NS - 1)
    def _zero_tail():
        pltpu.sync_copy(zsrc.at[pl.ds(0, _RTAIL)],
                        acc_sh.at[pl.ds(_RPT * _NS, _RTAIL)])

    plsc.subcore_barrier()

    def block_start(blk, slot):
        pltpu.async_copy(ecol_hbm.at[wid, blk], cslab.at[slot], esem)
        pltpu.async_copy(erow_hbm.at[wid, blk], rslab.at[slot], esem)
        pltpu.async_copy(eval_hbm.at[wid, blk], vslab.at[slot], esem)

    def block_wait(blk, slot):
        pltpu.make_async_copy(ecol_hbm.at[wid, blk], cslab.at[slot],
                              esem).wait()
        pltpu.make_async_copy(erow_hbm.at[wid, blk], rslab.at[slot],
                              esem).wait()
        pltpu.make_async_copy(eval_hbm.at[wid, blk], vslab.at[slot],
                              esem).wait()

    def gather_start(slot, i, b):
        pltpu.async_copy(ego_hbm.at[cslab.at[slot, i]], bufs.at[b],
                         gsems.at[b])

    def gather_wait(slot, i, b):
        pltpu.make_async_copy(ego_hbm.at[cslab.at[slot, i]], bufs.at[b],
                              gsems.at[b]).wait()

    def scale(slot, i, b):
        def group(g, gcarry):
            vv = vslab[slot, i, pl.ds(g * 16, 16)]
            for e in range(16):
                v16 = _splat(vv, e)
                row = g * 16 + e
                for d in range(_EMB // 16):
                    sl = pl.ds(d * 16, 16)
                    bufs[b, row, sl] = bufs[b, row, sl] * v16
            return gcarry

        lax.fori_loop(0, _K // 16, group, 0)

    pltpu.sync_copy(ecol_hbm.at[wid, 0], cslab.at[0])
    pltpu.sync_copy(erow_hbm.at[wid, 0], rslab.at[0])
    pltpu.sync_copy(eval_hbm.at[wid, 0], vslab.at[0])
    gather_start(0, 0, 0)

    def do_block(j, slot):
        @pl.when(j + 1 < _NB)
        def _stage_next():
            block_start(j + 1, 1 - slot)

        for i in range(_BB):
            b = i % 2
            gather_wait(slot, i, b)
            if i < _BB - 1:
                gather_start(slot, i + 1, 1 - b)
            else:
                @pl.when(j + 1 < _NB)
                def _cross_block():
                    block_wait(j + 1, 1 - slot)
                    gather_start(1 - slot, 0, 1 - b)

            scale(slot, i, b)
            pltpu.sync_copy(bufs.at[b], acc_sh.at[rslab.at[slot, i]],
                            add=True)

    def super_block(t, carry):
        do_block(t * 2, 0)
        do_block(t * 2 + 1, 1)
        return carry

    lax.fori_loop(0, _NB // 2, super_block, 0)
    for j in range(_NB - _NB % 2, _NB):
        do_block(j, j % 2)

    plsc.subcore_barrier()
    dstart = pl.multiple_of(sid * _RPT, 8)
    pltpu.sync_copy(acc_sh.at[pl.ds(dstart, _RPT)],
                    out_hbm.at[cid, pl.ds(dstart, _RPT)])

    @pl.when(sid == _NS - 1)
    def _dump_tail():
        pltpu.sync_copy(acc_sh.at[pl.ds(_RPT * _NS, _RTAIL)],
                        out_hbm.at[cid, pl.ds(_RPT * _NS, _RTAIL)])


@functools.cache
def _sc_spmv_build():
  return pl.kernel(
    _sc_spmv_body,
    out_type=jax.ShapeDtypeStruct((_NC, _N, _EMB), jnp.float32),
    mesh=plsc.VectorSubcoreMesh(core_axis_name="c", subcore_axis_name="s",
                                num_cores=_NC, num_subcores=_NS),
    scratch_types=[
        pltpu.VMEM((2, _BB, _K), jnp.int32),
        pltpu.VMEM((2, _BB, _K), jnp.int32),
        pltpu.VMEM((2, _BB, _K), jnp.float32),
        pltpu.VMEM((2, _K, _EMB), jnp.float32),
        pltpu.VMEM_SHARED((_N, _EMB), jnp.float32),
        pltpu.SemaphoreType.DMA((2,)),
        pltpu.SemaphoreType.DMA,
    ],
  )


def _sc_spmv(ego, ecol, erow, evalv):
    return _sc_spmv_build()(ego, ecol, erow, evalv)


def _leaky(x):
    return jnp.where(x >= 0, x, 0.01 * x)


def _tc_layer_body(parts_ref, ego_ref, wg_ref, bg_ref, wb_ref, bb_ref, out_ref):
    side = parts_ref[0] + parts_ref[1]
    ego = ego_ref[...]
    dn = (((1,), (1,)), ((), ()))
    s_pre = lax.dot_general(side, wg_ref[...], dn,
                            preferred_element_type=jnp.float32) + bg_ref[...]
    b_pre = lax.dot_general(ego * side, wb_ref[...], dn,
                            preferred_element_type=jnp.float32) + bb_ref[...]
    out_ref[...] = _leaky(s_pre) + _leaky(b_pre)


_LBLK = 2000


def _tc_layer(parts, ego, wg, bg, wb, bb):
    return pl.pallas_call(
        _tc_layer_body,
        grid=(_N // _LBLK,),
        in_specs=[
            pl.BlockSpec((_NC, _LBLK, _EMB), lambda i: (0, i, 0)),
            pl.BlockSpec((_LBLK, _EMB), lambda i: (i, 0)),
            pl.BlockSpec((_EMB, _EMB), lambda i: (0, 0)),
            pl.BlockSpec((1, _EMB), lambda i: (0, 0)),
            pl.BlockSpec((_EMB, _EMB), lambda i: (0, 0)),
            pl.BlockSpec((1, _EMB), lambda i: (0, 0)),
        ],
        out_specs=pl.BlockSpec((_LBLK, _EMB), lambda i: (i, 0)),
        out_shape=jax.ShapeDtypeStruct((_N, _EMB), jnp.float32),
    )(parts, ego, wg, bg.reshape(1, _EMB), wb, bb.reshape(1, _EMB))


def _tc_scores_body(u_ref, i_ref, out_ref):
    s = lax.dot_general(u_ref[...], i_ref[...], (((1,), (1,)), ((), ())),
                        preferred_element_type=jnp.float32)
    m = jnp.max(s, axis=1, keepdims=True)
    out_ref[...] = (s - m) - jnp.log(jnp.sum(jnp.exp(s - m), axis=1,
                                             keepdims=True))


_SBLK = 200


def _tc_scores(u_g, i_g):
    d = u_g.shape[1]
    return pl.pallas_call(
        _tc_scores_body,
        grid=(_NUM_USERS // _SBLK,),
        in_specs=[
            pl.BlockSpec((_SBLK, d), lambda i: (i, 0)),
            pl.BlockSpec((_NUM_ITEMS, d), lambda i: (0, 0)),
        ],
        out_specs=pl.BlockSpec((_SBLK, _NUM_ITEMS), lambda i: (i, 0)),
        out_shape=jax.ShapeDtypeStruct((_NUM_USERS, _NUM_ITEMS), jnp.float32),
    )(u_g, i_g)


def _pack_edges(edge_row, edge_col, edge_val):
    def pad(x, shape):
        return jnp.pad(x, (0, _NNZ_PAD - _NNZ)).reshape(shape)

    ecol = pad(edge_col, (_NW, _NB, _BB, _K))
    erow = pad(edge_row, (_NW, _NB, _BB, _K))
    evalv = pad(edge_val, (_NW, _NB, _BB, _K))
    return ecol, erow, evalv


def kernel(user_indices, item_indices, edge_row, edge_col, edge_val,
           user_table, item_table,
           W_gc0, b_gc0, W_bi0, b_bi0,
           W_gc1, b_gc1, W_bi1, b_bi1):
    ego0 = jnp.concatenate([user_table, item_table], axis=0)

    ecol, erow, evalv = _pack_edges(edge_row, edge_col, edge_val)

    parts0 = _sc_spmv(ego0, ecol, erow, evalv)
    ego1 = _tc_layer(parts0, ego0, W_gc0, b_gc0, W_bi0, b_bi0)

    parts1 = _sc_spmv(ego1, ecol, erow, evalv)
    ego2 = _tc_layer(parts1, ego1, W_gc1, b_gc1, W_bi1, b_bi1)

    u_g = jnp.concatenate(
        [ego0[:_NUM_USERS], ego1[:_NUM_USERS], ego2[:_NUM_USERS]], axis=1)
    i_g = jnp.concatenate(
        [ego0[_NUM_USERS:], ego1[_NUM_USERS:], ego2[_NUM_USERS:]], axis=1)
    return _tc_scores(u_g, i_g)


# speedup vs baseline: 1.0123x; 1.0123x over previous
import functools

import jax
import jax.numpy as jnp
from jax import lax
from jax.experimental import pallas as pl
from jax.experimental.pallas import tpu as pltpu
from jax.experimental.pallas import tpu_sc as plsc

_NUM_USERS = 2000
_NUM_ITEMS = 8000
_N = _NUM_USERS + _NUM_ITEMS
_EMB = 128
_NNZ = 320000

_NC = 2
_NS = 16
_NW = _NC * _NS
_K = 128
_BB = 16
_NB = 5
_CPW = _BB * _NB
_NNZ_PAD = _NW * _CPW * _K
_RPT = 624
_RTAIL = _N - _RPT * _NS
_ZROWS = _RPT // 3


def _splat(vv, e):
    idx = jnp.full((16, 1), e, jnp.int32)
    dn = lax.GatherDimensionNumbers(offset_dims=(), collapsed_slice_dims=(0,),
                                    start_index_map=(0,))
    return lax.gather(vv, idx, dn, (1,),
                      mode=lax.GatherScatterMode.PROMISE_IN_BOUNDS)


def _sc_spmv_body(ego_hbm, ecol_hbm, erow_hbm, eval_hbm, out_hbm,
                  cslab, rslab, vslab, bufs, acc_sh, gsems, esem):
    cid = lax.axis_index("c")
    sid = lax.axis_index("s")
    wid = cid * _NS + sid

    zero = jnp.zeros((16,), jnp.float32)

    def zrow(i, carry):
        for d in range(_EMB // 16):
            bufs[0, i, pl.ds(d * 16, 16)] = zero
        return carry

    lax.fori_loop(0, _K, zrow, 0)
    zsrc = bufs.at[0]
    zstart = pl.multiple_of(sid * _RPT, 8)
    for k in range(_RPT // _K):
        pltpu.sync_copy(zsrc, acc_sh.at[pl.ds(zstart + k * _K, _K)])
    pltpu.sync_copy(zsrc.at[pl.ds(0, _RPT % _K)],
                    acc_sh.at[pl.ds(zstart + _RPT - _RPT % _K, _RPT % _K)])

    @pl.when(sid == _NS - 1)
    def _zero_tail():
        pltpu.sync_copy(zsrc.at[pl.ds(0, _RTAIL)],
                        acc_sh.at[pl.ds(_RPT * _NS, _RTAIL)])

    plsc.subcore_barrier()

    def block_start(blk, slot):
        pltpu.async_copy(ecol_hbm.at[wid, blk], cslab.at[slot], esem)
        pltpu.async_copy(erow_hbm.at[wid, blk], rslab.at[slot], esem)
        pltpu.async_copy(eval_hbm.at[wid, blk], vslab.at[slot], esem)

    def block_wait(blk, slot):
        pltpu.make_async_copy(ecol_hbm.at[wid, blk], cslab.at[slot],
                              esem).wait()
        pltpu.make_async_copy(erow_hbm.at[wid, blk], rslab.at[slot],
                              esem).wait()
        pltpu.make_async_copy(eval_hbm.at[wid, blk], vslab.at[slot],
                              esem).wait()

    def gather_start(slot, i, b):
        pltpu.async_copy(ego_hbm.at[cslab.at[slot, i]], bufs.at[b],
                         gsems.at[b])

    def gather_wait(slot, i, b):
        pltpu.make_async_copy(ego_hbm.at[cslab.at[slot, i]], bufs.at[b],
                              gsems.at[b]).wait()

    def scale(slot, i, b):
        def group(g, gcarry):
            vv = vslab[slot, i, pl.ds(g * 16, 16)]
            for e in range(16):
                v16 = _splat(vv, e)
                row = g * 16 + e
                for d in range(_EMB // 16):
                    sl = pl.ds(d * 16, 16)
                    bufs[b, row, sl] = bufs[b, row, sl] * v16
            return gcarry

        lax.fori_loop(0, _K // 16, group, 0)

    pltpu.sync_copy(ecol_hbm.at[wid, 0], cslab.at[0])
    pltpu.sync_copy(erow_hbm.at[wid, 0], rslab.at[0])
    pltpu.sync_copy(eval_hbm.at[wid, 0], vslab.at[0])
    gather_start(0, 0, 0)

    def body(slot, i, b, start_next):
        gather_wait(slot, i, b)
        if start_next:
            gather_start(slot, i + 1, 1 - b)
        scale(slot, i, b)
        pltpu.sync_copy(bufs.at[b], acc_sh.at[rslab.at[slot, i]],
                        add=True)

    def do_block(j, slot):
        @pl.when(j + 1 < _NB)
        def _stage_next():
            block_start(j + 1, 1 - slot)

        def pair(p, carry):
            body(slot, p * 2, 0, True)
            body(slot, p * 2 + 1, 1, True)
            return carry

        lax.fori_loop(0, _BB // 2 - 1, pair, 0)
        body(slot, _BB - 2, 0, True)
        gather_wait(slot, _BB - 1, 1)

        @pl.when(j + 1 < _NB)
        def _cross_block():
            block_wait(j + 1, 1 - slot)
            gather_start(1 - slot, 0, 0)

        scale(slot, _BB - 1, 1)
        pltpu.sync_copy(bufs.at[1], acc_sh.at[rslab.at[slot, _BB - 1]],
                        add=True)

    def super_block(t, carry):
        do_block(t * 2, 0)
        do_block(t * 2 + 1, 1)
        return carry

    lax.fori_loop(0, _NB // 2, super_block, 0)
    for j in range(_NB - _NB % 2, _NB):
        do_block(j, j % 2)

    plsc.subcore_barrier()
    dstart = pl.multiple_of(sid * _RPT, 8)
    pltpu.sync_copy(acc_sh.at[pl.ds(dstart, _RPT)],
                    out_hbm.at[cid, pl.ds(dstart, _RPT)])

    @pl.when(sid == _NS - 1)
    def _dump_tail():
        pltpu.sync_copy(acc_sh.at[pl.ds(_RPT * _NS, _RTAIL)],
                        out_hbm.at[cid, pl.ds(_RPT * _NS, _RTAIL)])


@functools.cache
def _sc_spmv_build():
  return pl.kernel(
    _sc_spmv_body,
    out_type=jax.ShapeDtypeStruct((_NC, _N, _EMB), jnp.float32),
    mesh=plsc.VectorSubcoreMesh(core_axis_name="c", subcore_axis_name="s",
                                num_cores=_NC, num_subcores=_NS),
    scratch_types=[
        pltpu.VMEM((2, _BB, _K), jnp.int32),
        pltpu.VMEM((2, _BB, _K), jnp.int32),
        pltpu.VMEM((2, _BB, _K), jnp.float32),
        pltpu.VMEM((2, _K, _EMB), jnp.float32),
        pltpu.VMEM_SHARED((_N, _EMB), jnp.float32),
        pltpu.SemaphoreType.DMA((2,)),
        pltpu.SemaphoreType.DMA,
    ],
  )


def _sc_spmv(ego, ecol, erow, evalv):
    return _sc_spmv_build()(ego, ecol, erow, evalv)


def _leaky(x):
    return jnp.where(x >= 0, x, 0.01 * x)


def _tc_layer_body(parts_ref, ego_ref, wg_ref, bg_ref, wb_ref, bb_ref, out_ref):
    side = parts_ref[0] + parts_ref[1]
    ego = ego_ref[...]
    dn = (((1,), (1,)), ((), ()))
    s_pre = lax.dot_general(side, wg_ref[...], dn,
                            preferred_element_type=jnp.float32) + bg_ref[...]
    b_pre = lax.dot_general(ego * side, wb_ref[...], dn,
                            preferred_element_type=jnp.float32) + bb_ref[...]
    out_ref[...] = _leaky(s_pre) + _leaky(b_pre)


_LBLK = 2000


def _tc_layer(parts, ego, wg, bg, wb, bb):
    return pl.pallas_call(
        _tc_layer_body,
        grid=(_N // _LBLK,),
        in_specs=[
            pl.BlockSpec((_NC, _LBLK, _EMB), lambda i: (0, i, 0)),
            pl.BlockSpec((_LBLK, _EMB), lambda i: (i, 0)),
            pl.BlockSpec((_EMB, _EMB), lambda i: (0, 0)),
            pl.BlockSpec((1, _EMB), lambda i: (0, 0)),
            pl.BlockSpec((_EMB, _EMB), lambda i: (0, 0)),
            pl.BlockSpec((1, _EMB), lambda i: (0, 0)),
        ],
        out_specs=pl.BlockSpec((_LBLK, _EMB), lambda i: (i, 0)),
        out_shape=jax.ShapeDtypeStruct((_N, _EMB), jnp.float32),
    )(parts, ego, wg, bg.reshape(1, _EMB), wb, bb.reshape(1, _EMB))


def _tc_scores_body(u_ref, i_ref, out_ref):
    s = lax.dot_general(u_ref[...], i_ref[...], (((1,), (1,)), ((), ())),
                        preferred_element_type=jnp.float32)
    m = jnp.max(s, axis=1, keepdims=True)
    out_ref[...] = (s - m) - jnp.log(jnp.sum(jnp.exp(s - m), axis=1,
                                             keepdims=True))


_SBLK = 200


def _tc_scores(u_g, i_g):
    d = u_g.shape[1]
    return pl.pallas_call(
        _tc_scores_body,
        grid=(_NUM_USERS // _SBLK,),
        in_specs=[
            pl.BlockSpec((_SBLK, d), lambda i: (i, 0)),
            pl.BlockSpec((_NUM_ITEMS, d), lambda i: (0, 0)),
        ],
        out_specs=pl.BlockSpec((_SBLK, _NUM_ITEMS), lambda i: (i, 0)),
        out_shape=jax.ShapeDtypeStruct((_NUM_USERS, _NUM_ITEMS), jnp.float32),
    )(u_g, i_g)


def _pack_edges(edge_row, edge_col, edge_val):
    def pad(x, shape):
        return jnp.pad(x, (0, _NNZ_PAD - _NNZ)).reshape(shape)

    ecol = pad(edge_col, (_NW, _NB, _BB, _K))
    erow = pad(edge_row, (_NW, _NB, _BB, _K))
    evalv = pad(edge_val, (_NW, _NB, _BB, _K))
    return ecol, erow, evalv


def kernel(user_indices, item_indices, edge_row, edge_col, edge_val,
           user_table, item_table,
           W_gc0, b_gc0, W_bi0, b_bi0,
           W_gc1, b_gc1, W_bi1, b_bi1):
    ego0 = jnp.concatenate([user_table, item_table], axis=0)

    ecol, erow, evalv = _pack_edges(edge_row, edge_col, edge_val)

    parts0 = _sc_spmv(ego0, ecol, erow, evalv)
    ego1 = _tc_layer(parts0, ego0, W_gc0, b_gc0, W_bi0, b_bi0)

    parts1 = _sc_spmv(ego1, ecol, erow, evalv)
    ego2 = _tc_layer(parts1, ego1, W_gc1, b_gc1, W_bi1, b_bi1)

    u_g = jnp.concatenate(
        [ego0[:_NUM_USERS], ego1[:_NUM_USERS], ego2[:_NUM_USERS]], axis=1)
    i_g = jnp.concatenate(
        [ego0[_NUM_USERS:], ego1[_NUM_USERS:], ego2[_NUM_USERS:]], axis=1)
    return _tc_scores(u_g, i_g)


# speedup vs baseline: 1.7377x; 1.7167x over previous
import functools

import jax
import jax.numpy as jnp
from jax import lax
from jax.experimental import pallas as pl
from jax.experimental.pallas import tpu as pltpu
from jax.experimental.pallas import tpu_sc as plsc

_NUM_USERS = 2000
_NUM_ITEMS = 8000
_N = _NUM_USERS + _NUM_ITEMS
_EMB = 128
_NNZ = 320000

_NC = 2
_NS = 16
_NW = _NC * _NS
_K = 96
_BB = 15
_NB = 7
_CPW = _BB * _NB
_NNZ_PAD = _NW * _CPW * _K
_RPT = 624
_RTAIL = _N - _RPT * _NS
_ZROWS = _RPT // 3


def _splat(vv, e):
    idx = jnp.full((16, 1), e, jnp.int32)
    dn = lax.GatherDimensionNumbers(offset_dims=(), collapsed_slice_dims=(0,),
                                    start_index_map=(0,))
    return lax.gather(vv, idx, dn, (1,),
                      mode=lax.GatherScatterMode.PROMISE_IN_BOUNDS)


def _sc_spmv_body(ego_hbm, ecol_hbm, erow_hbm, eval_hbm, out_hbm,
                  cslab, rslab, vslab, bufs, acc_sh, gsems, esem, ssems):
    cid = lax.axis_index("c")
    sid = lax.axis_index("s")
    wid = cid * _NS + sid

    zero = jnp.zeros((16,), jnp.float32)

    def zrow(i, carry):
        for d in range(_EMB // 16):
            bufs[0, i, pl.ds(d * 16, 16)] = zero
        return carry

    lax.fori_loop(0, _K, zrow, 0)
    zsrc = bufs.at[0]
    zstart = pl.multiple_of(sid * _RPT, 8)
    for k in range(_RPT // _K):
        pltpu.sync_copy(zsrc, acc_sh.at[pl.ds(zstart + k * _K, _K)])
    pltpu.sync_copy(zsrc.at[pl.ds(0, _RPT % _K)],
                    acc_sh.at[pl.ds(zstart + _RPT - _RPT % _K, _RPT % _K)])

    @pl.when(sid == _NS - 1)
    def _zero_tail():
        pltpu.sync_copy(zsrc.at[pl.ds(0, _RTAIL)],
                        acc_sh.at[pl.ds(_RPT * _NS, _RTAIL)])

    plsc.subcore_barrier()

    def block_start(blk, slot):
        pltpu.async_copy(ecol_hbm.at[wid, blk], cslab.at[slot], esem)
        pltpu.async_copy(erow_hbm.at[wid, blk], rslab.at[slot], esem)
        pltpu.async_copy(eval_hbm.at[wid, blk], vslab.at[slot], esem)

    def block_wait(blk, slot):
        pltpu.make_async_copy(ecol_hbm.at[wid, blk], cslab.at[slot],
                              esem).wait()
        pltpu.make_async_copy(erow_hbm.at[wid, blk], rslab.at[slot],
                              esem).wait()
        pltpu.make_async_copy(eval_hbm.at[wid, blk], vslab.at[slot],
                              esem).wait()

    def gather_start(slot, i, b):
        pltpu.async_copy(ego_hbm.at[cslab.at[slot, i]], bufs.at[b],
                         gsems.at[b])

    def gather_wait(slot, i, b):
        pltpu.make_async_copy(ego_hbm.at[cslab.at[slot, i]], bufs.at[b],
                              gsems.at[b]).wait()

    def scale(slot, i, b):
        def group(g, gcarry):
            vv = vslab[slot, i, pl.ds(g * 16, 16)]
            for e in range(16):
                v16 = _splat(vv, e)
                row = g * 16 + e
                for d in range(_EMB // 16):
                    sl = pl.ds(d * 16, 16)
                    bufs[b, row, sl] = bufs[b, row, sl] * v16
            return gcarry

        lax.fori_loop(0, _K // 16, group, 0)

    def scatter_start(slot, i, b):
        pltpu.async_copy(bufs.at[b], acc_sh.at[rslab.at[slot, i]],
                         ssems.at[b], add=True)

    def scatter_wait_b(b):
        pltpu.make_async_copy(bufs.at[b], acc_sh.at[rslab.at[0, 0]],
                              ssems.at[b]).wait()

    pltpu.sync_copy(ecol_hbm.at[wid, 0], cslab.at[0])
    pltpu.sync_copy(erow_hbm.at[wid, 0], rslab.at[0])
    pltpu.sync_copy(eval_hbm.at[wid, 0], vslab.at[0])
    gather_start(0, 0, 0)

    def phase(j, slot, i, q, cross):
        c = j * _BB + i

        @pl.when(c >= 2)
        def _reap():
            scatter_wait_b((q + 1) % 3)

        if not cross:
            gather_start(slot, i + 1, (q + 1) % 3)
        else:
            @pl.when(j + 1 < _NB)
            def _cross_block():
                block_wait(j + 1, 1 - slot)
                gather_start(1 - slot, 0, 0)

        gather_wait(slot, i, q)
        scale(slot, i, q)
        scatter_start(slot, i, q)

    def do_block(j, slot):
        def triple(p, carry):
            @pl.when((p == 0) & (j + 1 < _NB))
            def _stage_next():
                block_start(j + 1, 1 - slot)

            for q in range(3):
                phase(j, slot, p * 3 + q, q, False)
            return carry

        lax.fori_loop(0, _BB // 3 - 1, triple, 0)
        phase(j, slot, _BB - 3, 0, False)
        phase(j, slot, _BB - 2, 1, False)
        phase(j, slot, _BB - 1, 2, True)

    def super_block(t, carry):
        do_block(t * 2, 0)
        do_block(t * 2 + 1, 1)
        return carry

    lax.fori_loop(0, _NB // 2, super_block, 0)
    for j in range(_NB - _NB % 2, _NB):
        do_block(j, j % 2)

    scatter_wait_b(1)
    scatter_wait_b(2)

    plsc.subcore_barrier()
    dstart = pl.multiple_of(sid * _RPT, 8)
    pltpu.sync_copy(acc_sh.at[pl.ds(dstart, _RPT)],
                    out_hbm.at[cid, pl.ds(dstart, _RPT)])

    @pl.when(sid == _NS - 1)
    def _dump_tail():
        pltpu.sync_copy(acc_sh.at[pl.ds(_RPT * _NS, _RTAIL)],
                        out_hbm.at[cid, pl.ds(_RPT * _NS, _RTAIL)])


@functools.cache
def _sc_spmv_build():
  return pl.kernel(
    _sc_spmv_body,
    out_type=jax.ShapeDtypeStruct((_NC, _N, _EMB), jnp.float32),
    mesh=plsc.VectorSubcoreMesh(core_axis_name="c", subcore_axis_name="s",
                                num_cores=_NC, num_subcores=_NS),
    scratch_types=[
        pltpu.VMEM((2, _BB, _K), jnp.int32),
        pltpu.VMEM((2, _BB, _K), jnp.int32),
        pltpu.VMEM((2, _BB, _K), jnp.float32),
        pltpu.VMEM((3, _K, _EMB), jnp.float32),
        pltpu.VMEM_SHARED((_N, _EMB), jnp.float32),
        pltpu.SemaphoreType.DMA((3,)),
        pltpu.SemaphoreType.DMA,
        pltpu.SemaphoreType.DMA((3,)),
    ],
  )


def _sc_spmv(ego, ecol, erow, evalv):
    return _sc_spmv_build()(ego, ecol, erow, evalv)


def _leaky(x):
    return jnp.where(x >= 0, x, 0.01 * x)


def _tc_layer_body(parts_ref, ego_ref, wg_ref, bg_ref, wb_ref, bb_ref, out_ref):
    side = parts_ref[0] + parts_ref[1]
    ego = ego_ref[...]
    dn = (((1,), (1,)), ((), ()))
    s_pre = lax.dot_general(side, wg_ref[...], dn,
                            preferred_element_type=jnp.float32) + bg_ref[...]
    b_pre = lax.dot_general(ego * side, wb_ref[...], dn,
                            preferred_element_type=jnp.float32) + bb_ref[...]
    out_ref[...] = _leaky(s_pre) + _leaky(b_pre)


_LBLK = 2000


def _tc_layer(parts, ego, wg, bg, wb, bb):
    return pl.pallas_call(
        _tc_layer_body,
        grid=(_N // _LBLK,),
        in_specs=[
            pl.BlockSpec((_NC, _LBLK, _EMB), lambda i: (0, i, 0)),
            pl.BlockSpec((_LBLK, _EMB), lambda i: (i, 0)),
            pl.BlockSpec((_EMB, _EMB), lambda i: (0, 0)),
            pl.BlockSpec((1, _EMB), lambda i: (0, 0)),
            pl.BlockSpec((_EMB, _EMB), lambda i: (0, 0)),
            pl.BlockSpec((1, _EMB), lambda i: (0, 0)),
        ],
        out_specs=pl.BlockSpec((_LBLK, _EMB), lambda i: (i, 0)),
        out_shape=jax.ShapeDtypeStruct((_N, _EMB), jnp.float32),
    )(parts, ego, wg, bg.reshape(1, _EMB), wb, bb.reshape(1, _EMB))


def _tc_scores_body(u_ref, i_ref, out_ref):
    s = lax.dot_general(u_ref[...], i_ref[...], (((1,), (1,)), ((), ())),
                        preferred_element_type=jnp.float32)
    m = jnp.max(s, axis=1, keepdims=True)
    out_ref[...] = (s - m) - jnp.log(jnp.sum(jnp.exp(s - m), axis=1,
                                             keepdims=True))


_SBLK = 200


def _tc_scores(u_g, i_g):
    d = u_g.shape[1]
    return pl.pallas_call(
        _tc_scores_body,
        grid=(_NUM_USERS // _SBLK,),
        in_specs=[
            pl.BlockSpec((_SBLK, d), lambda i: (i, 0)),
            pl.BlockSpec((_NUM_ITEMS, d), lambda i: (0, 0)),
        ],
        out_specs=pl.BlockSpec((_SBLK, _NUM_ITEMS), lambda i: (i, 0)),
        out_shape=jax.ShapeDtypeStruct((_NUM_USERS, _NUM_ITEMS), jnp.float32),
    )(u_g, i_g)


def _pack_edges(edge_row, edge_col, edge_val):
    def pad(x, shape):
        return jnp.pad(x, (0, _NNZ_PAD - _NNZ)).reshape(shape)

    ecol = pad(edge_col, (_NW, _NB, _BB, _K))
    erow = pad(edge_row, (_NW, _NB, _BB, _K))
    evalv = pad(edge_val, (_NW, _NB, _BB, _K))
    return ecol, erow, evalv


def kernel(user_indices, item_indices, edge_row, edge_col, edge_val,
           user_table, item_table,
           W_gc0, b_gc0, W_bi0, b_bi0,
           W_gc1, b_gc1, W_bi1, b_bi1):
    ego0 = jnp.concatenate([user_table, item_table], axis=0)

    ecol, erow, evalv = _pack_edges(edge_row, edge_col, edge_val)

    parts0 = _sc_spmv(ego0, ecol, erow, evalv)
    ego1 = _tc_layer(parts0, ego0, W_gc0, b_gc0, W_bi0, b_bi0)

    parts1 = _sc_spmv(ego1, ecol, erow, evalv)
    ego2 = _tc_layer(parts1, ego1, W_gc1, b_gc1, W_bi1, b_bi1)

    u_g = jnp.concatenate(
        [ego0[:_NUM_USERS], ego1[:_NUM_USERS], ego2[:_NUM_USERS]], axis=1)
    i_g = jnp.concatenate(
        [ego0[_NUM_USERS:], ego1[_NUM_USERS:], ego2[_NUM_USERS:]], axis=1)
    return _tc_scores(u_g, i_g)


# speedup vs baseline: 1.7383x; 1.0003x over previous
import functools

import jax
import jax.numpy as jnp
from jax import lax
from jax.experimental import pallas as pl
from jax.experimental.pallas import tpu as pltpu
from jax.experimental.pallas import tpu_sc as plsc

_NUM_USERS = 2000
_NUM_ITEMS = 8000
_N = _NUM_USERS + _NUM_ITEMS
_EMB = 128
_NNZ = 320000

_NC = 2
_NS = 16
_NW = _NC * _NS
_K = 96
_BB = 15
_NB = 7
_CPW = _BB * _NB
_NNZ_PAD = _NW * _CPW * _K
_RPT = 624
_RTAIL = _N - _RPT * _NS
_ZROWS = _RPT // 3


def _splat(vv, e):
    idx = jnp.full((16, 1), e, jnp.int32)
    dn = lax.GatherDimensionNumbers(offset_dims=(), collapsed_slice_dims=(0,),
                                    start_index_map=(0,))
    return lax.gather(vv, idx, dn, (1,),
                      mode=lax.GatherScatterMode.PROMISE_IN_BOUNDS)


def _sc_spmv_body(ego_hbm, ecol_hbm, erow_hbm, eval_hbm, out_hbm,
                  cslab, rslab, vslab, bufs, acc_sh, gsems, esem, ssems):
    cid = lax.axis_index("c")
    sid = lax.axis_index("s")
    wid = cid * _NS + sid

    zero = jnp.zeros((16,), jnp.float32)

    def zrow(i, carry):
        for d in range(_EMB // 16):
            bufs[0, i, pl.ds(d * 16, 16)] = zero
        return carry

    lax.fori_loop(0, _K, zrow, 0)
    zsrc = bufs.at[0]
    zstart = pl.multiple_of(sid * _RPT, 8)
    for k in range(_RPT // _K):
        pltpu.sync_copy(zsrc, acc_sh.at[pl.ds(zstart + k * _K, _K)])
    pltpu.sync_copy(zsrc.at[pl.ds(0, _RPT % _K)],
                    acc_sh.at[pl.ds(zstart + _RPT - _RPT % _K, _RPT % _K)])

    @pl.when(sid == _NS - 1)
    def _zero_tail():
        pltpu.sync_copy(zsrc.at[pl.ds(0, _RTAIL)],
                        acc_sh.at[pl.ds(_RPT * _NS, _RTAIL)])

    plsc.subcore_barrier()

    def block_start(blk, slot):
        pltpu.async_copy(ecol_hbm.at[wid, blk], cslab.at[slot], esem)
        pltpu.async_copy(erow_hbm.at[wid, blk], rslab.at[slot], esem)
        pltpu.async_copy(eval_hbm.at[wid, blk], vslab.at[slot], esem)

    def block_wait(blk, slot):
        pltpu.make_async_copy(ecol_hbm.at[wid, blk], cslab.at[slot],
                              esem).wait()
        pltpu.make_async_copy(erow_hbm.at[wid, blk], rslab.at[slot],
                              esem).wait()
        pltpu.make_async_copy(eval_hbm.at[wid, blk], vslab.at[slot],
                              esem).wait()

    def gather_start(slot, i, b):
        for h in range(2):
            pltpu.async_copy(
                ego_hbm.at[cslab.at[slot, i, pl.ds(h * (_K // 2), _K // 2)]],
                bufs.at[b, pl.ds(h * (_K // 2), _K // 2)], gsems.at[b])

    def gather_wait(slot, i, b):
        for h in range(2):
            pltpu.make_async_copy(
                ego_hbm.at[cslab.at[slot, i, pl.ds(h * (_K // 2), _K // 2)]],
                bufs.at[b, pl.ds(h * (_K // 2), _K // 2)], gsems.at[b]).wait()

    def scale(slot, i, b):
        def group(g, gcarry):
            vv = vslab[slot, i, pl.ds(g * 16, 16)]
            for e in range(16):
                v16 = _splat(vv, e)
                row = g * 16 + e
                for d in range(_EMB // 16):
                    sl = pl.ds(d * 16, 16)
                    bufs[b, row, sl] = bufs[b, row, sl] * v16
            return gcarry

        lax.fori_loop(0, _K // 16, group, 0)

    def scatter_start(slot, i, b):
        pltpu.async_copy(bufs.at[b], acc_sh.at[rslab.at[slot, i]],
                         ssems.at[b], add=True)

    def scatter_wait_b(b):
        pltpu.make_async_copy(bufs.at[b], acc_sh.at[rslab.at[0, 0]],
                              ssems.at[b]).wait()

    pltpu.sync_copy(ecol_hbm.at[wid, 0], cslab.at[0])
    pltpu.sync_copy(erow_hbm.at[wid, 0], rslab.at[0])
    pltpu.sync_copy(eval_hbm.at[wid, 0], vslab.at[0])
    gather_start(0, 0, 0)

    def phase(j, slot, i, q, cross):
        c = j * _BB + i

        @pl.when(c >= 2)
        def _reap():
            scatter_wait_b((q + 1) % 3)

        if not cross:
            gather_start(slot, i + 1, (q + 1) % 3)
        else:
            @pl.when(j + 1 < _NB)
            def _cross_block():
                block_wait(j + 1, 1 - slot)
                gather_start(1 - slot, 0, 0)

        gather_wait(slot, i, q)
        scale(slot, i, q)
        scatter_start(slot, i, q)

    def do_block(j, slot):
        def triple(p, carry):
            @pl.when((p == 0) & (j + 1 < _NB))
            def _stage_next():
                block_start(j + 1, 1 - slot)

            for q in range(3):
                phase(j, slot, p * 3 + q, q, False)
            return carry

        lax.fori_loop(0, _BB // 3 - 1, triple, 0)
        phase(j, slot, _BB - 3, 0, False)
        phase(j, slot, _BB - 2, 1, False)
        phase(j, slot, _BB - 1, 2, True)

    def super_block(t, carry):
        do_block(t * 2, 0)
        do_block(t * 2 + 1, 1)
        return carry

    lax.fori_loop(0, _NB // 2, super_block, 0)
    for j in range(_NB - _NB % 2, _NB):
        do_block(j, j % 2)

    scatter_wait_b(1)
    scatter_wait_b(2)

    plsc.subcore_barrier()
    dstart = pl.multiple_of(sid * _RPT, 8)
    pltpu.sync_copy(acc_sh.at[pl.ds(dstart, _RPT)],
                    out_hbm.at[cid, pl.ds(dstart, _RPT)])

    @pl.when(sid == _NS - 1)
    def _dump_tail():
        pltpu.sync_copy(acc_sh.at[pl.ds(_RPT * _NS, _RTAIL)],
                        out_hbm.at[cid, pl.ds(_RPT * _NS, _RTAIL)])


@functools.cache
def _sc_spmv_build():
  return pl.kernel(
    _sc_spmv_body,
    out_type=jax.ShapeDtypeStruct((_NC, _N, _EMB), jnp.float32),
    mesh=plsc.VectorSubcoreMesh(core_axis_name="c", subcore_axis_name="s",
                                num_cores=_NC, num_subcores=_NS),
    scratch_types=[
        pltpu.VMEM((2, _BB, _K), jnp.int32),
        pltpu.VMEM((2, _BB, _K), jnp.int32),
        pltpu.VMEM((2, _BB, _K), jnp.float32),
        pltpu.VMEM((3, _K, _EMB), jnp.float32),
        pltpu.VMEM_SHARED((_N, _EMB), jnp.float32),
        pltpu.SemaphoreType.DMA((3,)),
        pltpu.SemaphoreType.DMA,
        pltpu.SemaphoreType.DMA((3,)),
    ],
  )


def _sc_spmv(ego, ecol, erow, evalv):
    return _sc_spmv_build()(ego, ecol, erow, evalv)


def _leaky(x):
    return jnp.where(x >= 0, x, 0.01 * x)


def _tc_layer_body(parts_ref, ego_ref, wg_ref, bg_ref, wb_ref, bb_ref, out_ref):
    side = parts_ref[0] + parts_ref[1]
    ego = ego_ref[...]
    dn = (((1,), (1,)), ((), ()))
    s_pre = lax.dot_general(side, wg_ref[...], dn,
                            preferred_element_type=jnp.float32) + bg_ref[...]
    b_pre = lax.dot_general(ego * side, wb_ref[...], dn,
                            preferred_element_type=jnp.float32) + bb_ref[...]
    out_ref[...] = _leaky(s_pre) + _leaky(b_pre)


_LBLK = 2000


def _tc_layer(parts, ego, wg, bg, wb, bb):
    return pl.pallas_call(
        _tc_layer_body,
        grid=(_N // _LBLK,),
        in_specs=[
            pl.BlockSpec((_NC, _LBLK, _EMB), lambda i: (0, i, 0)),
            pl.BlockSpec((_LBLK, _EMB), lambda i: (i, 0)),
            pl.BlockSpec((_EMB, _EMB), lambda i: (0, 0)),
            pl.BlockSpec((1, _EMB), lambda i: (0, 0)),
            pl.BlockSpec((_EMB, _EMB), lambda i: (0, 0)),
            pl.BlockSpec((1, _EMB), lambda i: (0, 0)),
        ],
        out_specs=pl.BlockSpec((_LBLK, _EMB), lambda i: (i, 0)),
        out_shape=jax.ShapeDtypeStruct((_N, _EMB), jnp.float32),
    )(parts, ego, wg, bg.reshape(1, _EMB), wb, bb.reshape(1, _EMB))


def _tc_scores_body(u_ref, i_ref, out_ref):
    s = lax.dot_general(u_ref[...], i_ref[...], (((1,), (1,)), ((), ())),
                        preferred_element_type=jnp.float32)
    m = jnp.max(s, axis=1, keepdims=True)
    out_ref[...] = (s - m) - jnp.log(jnp.sum(jnp.exp(s - m), axis=1,
                                             keepdims=True))


_SBLK = 200


def _tc_scores(u_g, i_g):
    d = u_g.shape[1]
    return pl.pallas_call(
        _tc_scores_body,
        grid=(_NUM_USERS // _SBLK,),
        in_specs=[
            pl.BlockSpec((_SBLK, d), lambda i: (i, 0)),
            pl.BlockSpec((_NUM_ITEMS, d), lambda i: (0, 0)),
        ],
        out_specs=pl.BlockSpec((_SBLK, _NUM_ITEMS), lambda i: (i, 0)),
        out_shape=jax.ShapeDtypeStruct((_NUM_USERS, _NUM_ITEMS), jnp.float32),
    )(u_g, i_g)


def _pack_edges(edge_row, edge_col, edge_val):
    def pad(x, shape):
        return jnp.pad(x, (0, _NNZ_PAD - _NNZ)).reshape(shape)

    ecol = pad(edge_col, (_NW, _NB, _BB, _K))
    erow = pad(edge_row, (_NW, _NB, _BB, _K))
    evalv = pad(edge_val, (_NW, _NB, _BB, _K))
    return ecol, erow, evalv


def kernel(user_indices, item_indices, edge_row, edge_col, edge_val,
           user_table, item_table,
           W_gc0, b_gc0, W_bi0, b_bi0,
           W_gc1, b_gc1, W_bi1, b_bi1):
    ego0 = jnp.concatenate([user_table, item_table], axis=0)

    ecol, erow, evalv = _pack_edges(edge_row, edge_col, edge_val)

    parts0 = _sc_spmv(ego0, ecol, erow, evalv)
    ego1 = _tc_layer(parts0, ego0, W_gc0, b_gc0, W_bi0, b_bi0)

    parts1 = _sc_spmv(ego1, ecol, erow, evalv)
    ego2 = _tc_layer(parts1, ego1, W_gc1, b_gc1, W_bi1, b_bi1)

    u_g = jnp.concatenate(
        [ego0[:_NUM_USERS], ego1[:_NUM_USERS], ego2[:_NUM_USERS]], axis=1)
    i_g = jnp.concatenate(
        [ego0[_NUM_USERS:], ego1[_NUM_USERS:], ego2[_NUM_USERS:]], axis=1)
    return _tc_scores(u_g, i_g)


# speedup vs baseline: 1.7393x; 1.0006x over previous
import functools

import jax
import jax.numpy as jnp
from jax import lax
from jax.experimental import pallas as pl
from jax.experimental.pallas import tpu as pltpu
from jax.experimental.pallas import tpu_sc as plsc

_NUM_USERS = 2000
_NUM_ITEMS = 8000
_N = _NUM_USERS + _NUM_ITEMS
_EMB = 128
_NNZ = 320000

_NC = 2
_NS = 16
_NW = _NC * _NS
_K = 96
_BB = 15
_NB = 7
_CPW = _BB * _NB
_NNZ_PAD = _NW * _CPW * _K
_RPT = 624
_RTAIL = _N - _RPT * _NS
_ZROWS = _RPT // 3


def _splat(vv, e):
    idx = jnp.full((16, 1), e, jnp.int32)
    dn = lax.GatherDimensionNumbers(offset_dims=(), collapsed_slice_dims=(0,),
                                    start_index_map=(0,))
    return lax.gather(vv, idx, dn, (1,),
                      mode=lax.GatherScatterMode.PROMISE_IN_BOUNDS)


def _sc_spmv_body(ego_hbm, ecol_hbm, erow_hbm, eval_hbm, out_hbm,
                  cslab, rslab, vslab, bufs, acc_sh, gsems, esem, ssems):
    cid = lax.axis_index("c")
    sid = lax.axis_index("s")
    wid = cid * _NS + sid

    zero = jnp.zeros((16,), jnp.float32)

    def zrow(i, carry):
        for d in range(_EMB // 16):
            bufs[0, i, pl.ds(d * 16, 16)] = zero
        return carry

    lax.fori_loop(0, _K, zrow, 0)
    zsrc = bufs.at[0]
    zstart = pl.multiple_of(sid * _RPT, 8)
    for k in range(_RPT // _K):
        pltpu.sync_copy(zsrc, acc_sh.at[pl.ds(zstart + k * _K, _K)])
    pltpu.sync_copy(zsrc.at[pl.ds(0, _RPT % _K)],
                    acc_sh.at[pl.ds(zstart + _RPT - _RPT % _K, _RPT % _K)])

    @pl.when(sid == _NS - 1)
    def _zero_tail():
        pltpu.sync_copy(zsrc.at[pl.ds(0, _RTAIL)],
                        acc_sh.at[pl.ds(_RPT * _NS, _RTAIL)])

    plsc.subcore_barrier()

    def block_start(blk, slot):
        pltpu.async_copy(ecol_hbm.at[wid, blk], cslab.at[slot], esem)
        pltpu.async_copy(erow_hbm.at[wid, blk], rslab.at[slot], esem)
        pltpu.async_copy(eval_hbm.at[wid, blk], vslab.at[slot], esem)

    def block_wait(blk, slot):
        pltpu.make_async_copy(ecol_hbm.at[wid, blk], cslab.at[slot],
                              esem).wait()
        pltpu.make_async_copy(erow_hbm.at[wid, blk], rslab.at[slot],
                              esem).wait()
        pltpu.make_async_copy(eval_hbm.at[wid, blk], vslab.at[slot],
                              esem).wait()

    def gather_start(slot, i, b):
        pltpu.async_copy(ego_hbm.at[cslab.at[slot, i]], bufs.at[b],
                         gsems.at[b])

    def gather_wait(slot, i, b):
        pltpu.make_async_copy(ego_hbm.at[cslab.at[slot, i]], bufs.at[b],
                              gsems.at[b]).wait()

    def scale(slot, i, b):
        def group(g, gcarry):
            vv = vslab[slot, i, pl.ds(g * 16, 16)]
            for e in range(16):
                v16 = _splat(vv, e)
                row = g * 16 + e
                for d in range(_EMB // 16):
                    sl = pl.ds(d * 16, 16)
                    bufs[b, row, sl] = bufs[b, row, sl] * v16
            return gcarry

        lax.fori_loop(0, _K // 16, group, 0)

    def scatter_start(slot, i, b):
        pltpu.async_copy(bufs.at[b], acc_sh.at[rslab.at[slot, i]],
                         ssems.at[b], add=True)

    def scatter_wait_b(b):
        pltpu.make_async_copy(bufs.at[b], acc_sh.at[rslab.at[0, 0]],
                              ssems.at[b]).wait()

    pltpu.sync_copy(ecol_hbm.at[wid, 0], cslab.at[0])
    pltpu.sync_copy(erow_hbm.at[wid, 0], rslab.at[0])
    pltpu.sync_copy(eval_hbm.at[wid, 0], vslab.at[0])
    gather_start(0, 0, 0)

    def phase(j, slot, i, q, cross):
        c = j * _BB + i

        @pl.when(c >= 2)
        def _reap():
            scatter_wait_b((q + 1) % 3)

        if not cross:
            gather_start(slot, i + 1, (q + 1) % 3)
        else:
            @pl.when(j + 1 < _NB)
            def _cross_block():
                block_wait(j + 1, 1 - slot)
                gather_start(1 - slot, 0, 0)

        gather_wait(slot, i, q)
        scale(slot, i, q)
        scatter_start(slot, i, q)

    def do_block(j, slot):
        def triple(p, carry):
            @pl.when((p == 0) & (j + 1 < _NB))
            def _stage_next():
                block_start(j + 1, 1 - slot)

            for q in range(3):
                phase(j, slot, p * 3 + q, q, False)
            return carry

        lax.fori_loop(0, _BB // 3 - 1, triple, 0)
        phase(j, slot, _BB - 3, 0, False)
        phase(j, slot, _BB - 2, 1, False)
        phase(j, slot, _BB - 1, 2, True)

    def super_block(t, carry):
        do_block(t * 2, 0)
        do_block(t * 2 + 1, 1)
        return carry

    lax.fori_loop(0, _NB // 2, super_block, 0)
    for j in range(_NB - _NB % 2, _NB):
        do_block(j, j % 2)

    scatter_wait_b(1)
    scatter_wait_b(2)

    plsc.subcore_barrier()
    dstart = pl.multiple_of(sid * _RPT, 8)
    pltpu.sync_copy(acc_sh.at[pl.ds(dstart, _RPT)],
                    out_hbm.at[cid, pl.ds(dstart, _RPT)])

    @pl.when(sid == _NS - 1)
    def _dump_tail():
        pltpu.sync_copy(acc_sh.at[pl.ds(_RPT * _NS, _RTAIL)],
                        out_hbm.at[cid, pl.ds(_RPT * _NS, _RTAIL)])


@functools.cache
def _sc_spmv_build():
  return pl.kernel(
    _sc_spmv_body,
    out_type=jax.ShapeDtypeStruct((_NC, _N, _EMB), jnp.float32),
    mesh=plsc.VectorSubcoreMesh(core_axis_name="c", subcore_axis_name="s",
                                num_cores=_NC, num_subcores=_NS),
    scratch_types=[
        pltpu.VMEM((2, _BB, _K), jnp.int32),
        pltpu.VMEM((2, _BB, _K), jnp.int32),
        pltpu.VMEM((2, _BB, _K), jnp.float32),
        pltpu.VMEM((3, _K, _EMB), jnp.float32),
        pltpu.VMEM_SHARED((_N, _EMB), jnp.float32),
        pltpu.SemaphoreType.DMA((3,)),
        pltpu.SemaphoreType.DMA,
        pltpu.SemaphoreType.DMA((3,)),
    ],
  )


def _sc_spmv(ego, ecol, erow, evalv):
    return _sc_spmv_build()(ego, ecol, erow, evalv)


def _leaky(x):
    return jnp.where(x >= 0, x, 0.01 * x)


def _tc_layer_body(parts_ref, ego_ref, wg_ref, bg_ref, wb_ref, bb_ref, out_ref):
    side = parts_ref[0] + parts_ref[1]
    ego = ego_ref[...]
    dn = (((1,), (1,)), ((), ()))
    s_pre = lax.dot_general(side, wg_ref[...], dn,
                            preferred_element_type=jnp.float32) + bg_ref[...]
    b_pre = lax.dot_general(ego * side, wb_ref[...], dn,
                            preferred_element_type=jnp.float32) + bb_ref[...]
    out_ref[...] = _leaky(s_pre) + _leaky(b_pre)


_LBLK = 2000


def _tc_layer(parts, ego, wg, bg, wb, bb):
    return pl.pallas_call(
        _tc_layer_body,
        grid=(_N // _LBLK,),
        in_specs=[
            pl.BlockSpec((_NC, _LBLK, _EMB), lambda i: (0, i, 0)),
            pl.BlockSpec((_LBLK, _EMB), lambda i: (i, 0)),
            pl.BlockSpec((_EMB, _EMB), lambda i: (0, 0)),
            pl.BlockSpec((1, _EMB), lambda i: (0, 0)),
            pl.BlockSpec((_EMB, _EMB), lambda i: (0, 0)),
            pl.BlockSpec((1, _EMB), lambda i: (0, 0)),
        ],
        out_specs=pl.BlockSpec((_LBLK, _EMB), lambda i: (i, 0)),
        out_shape=jax.ShapeDtypeStruct((_N, _EMB), jnp.float32),
    )(parts, ego, wg, bg.reshape(1, _EMB), wb, bb.reshape(1, _EMB))


def _tc_scores_body(u_ref, i_ref, out_ref):
    s = lax.dot_general(u_ref[...], i_ref[...], (((1,), (1,)), ((), ())),
                        preferred_element_type=jnp.float32)
    m = jnp.max(s, axis=1, keepdims=True)
    out_ref[...] = (s - m) - jnp.log(jnp.sum(jnp.exp(s - m), axis=1,
                                             keepdims=True))


_SBLK = 200


def _tc_scores(u_g, i_g):
    d = u_g.shape[1]
    return pl.pallas_call(
        _tc_scores_body,
        grid=(_NUM_USERS // _SBLK,),
        in_specs=[
            pl.BlockSpec((_SBLK, d), lambda i: (i, 0)),
            pl.BlockSpec((_NUM_ITEMS, d), lambda i: (0, 0)),
        ],
        out_specs=pl.BlockSpec((_SBLK, _NUM_ITEMS), lambda i: (i, 0)),
        out_shape=jax.ShapeDtypeStruct((_NUM_USERS, _NUM_ITEMS), jnp.float32),
    )(u_g, i_g)


def _pack_edges(edge_row, edge_col, edge_val):
    def pad(x, shape):
        return jnp.pad(x, (0, _NNZ_PAD - _NNZ)).reshape(shape)

    ecol = pad(edge_col, (_NW, _NB, _BB, _K))
    erow = pad(edge_row, (_NW, _NB, _BB, _K))
    evalv = pad(edge_val, (_NW, _NB, _BB, _K))
    return ecol, erow, evalv


def kernel(user_indices, item_indices, edge_row, edge_col, edge_val,
           user_table, item_table,
           W_gc0, b_gc0, W_bi0, b_bi0,
           W_gc1, b_gc1, W_bi1, b_bi1):
    ego0 = jnp.concatenate([user_table, item_table], axis=0)

    ecol, erow, evalv = _pack_edges(edge_row, edge_col, edge_val)

    parts0 = _sc_spmv(ego0, ecol, erow, evalv)
    ego1 = _tc_layer(parts0, ego0, W_gc0, b_gc0, W_bi0, b_bi0)

    parts1 = _sc_spmv(ego1, ecol, erow, evalv)
    ego2 = _tc_layer(parts1, ego1, W_gc1, b_gc1, W_bi1, b_bi1)

    u_g = jnp.concatenate(
        [ego0[:_NUM_USERS], ego1[:_NUM_USERS], ego2[:_NUM_USERS]], axis=1)
    i_g = jnp.concatenate(
        [ego0[_NUM_USERS:], ego1[_NUM_USERS:], ego2[_NUM_USERS:]], axis=1)
    return _tc_scores(u_g, i_g)
